# Initial kernel scaffold; baseline (speedup 1.0000x reference)
import jax
import jax.numpy as jnp
from jax.experimental import pallas as pl


def kernel(coords, vector_field):
    raise NotImplementedError("write your pallas kernel here")


# speedup vs baseline: 32.4944x; 32.4944x over previous
import functools

import jax
import jax.numpy as jnp
from jax import lax
from jax.experimental import pallas as pl
from jax.experimental.pallas import tpu as pltpu
from jax.experimental.pallas import tpu_sc as plsc

H, W, FD = 1024, 1024, 2
W2 = W + 1
NC, NS, L = 2, 16, 16
NW = NC * NS

N = 16384 * 200
PER_W = N // NW
C = 2048
CHUNKS = PER_W // C

PV = 16 * 65728
STAGE = PV // NS


def _sc_body(coords_hbm, table_hbm, out_hbm,
             shared, coords_v, i00_v, i01_v, i10_v, i11_v, wx_v, wy_v,
             r00_v, r01_v, r10_v, r11_v, out_v, sem):
    cid = lax.axis_index("c")
    sid = lax.axis_index("s")
    wid = sid * NC + cid
    base_coord = wid * PER_W

    pltpu.sync_copy(table_hbm.at[pl.ds(sid * STAGE, STAGE)],
                    shared.at[pl.ds(sid * STAGE, STAGE)])
    plsc.subcore_barrier()

    def chunk_body(g, carry):
        cbase = base_coord + g * C
        fbase = cbase * 2
        pltpu.sync_copy(coords_hbm.at[pl.ds(fbase, 2 * C)], coords_v)

        def idx_body(i, carry2):
            iota = lax.iota(jnp.int32, L)
            xi = iota * 2 + i * (2 * L)
            x = plsc.load_gather(coords_v, [xi]) * float(W - 1)
            y = plsc.load_gather(coords_v, [xi + 1]) * float(H - 1)
            x0 = x.astype(jnp.int32)
            y0 = y.astype(jnp.int32)
            wx = x - x0.astype(jnp.float32)
            wy = y - y0.astype(jnp.float32)
            idx = y0 * W2 + x0
            b = i * L
            i00_v[pl.ds(b, L)] = idx
            i01_v[pl.ds(b, L)] = idx + 1
            i10_v[pl.ds(b, L)] = idx + W2
            i11_v[pl.ds(b, L)] = idx + (W2 + 1)
            wx_v[pl.ds(b, L)] = wx
            wy_v[pl.ds(b, L)] = wy
            return carry2

        lax.fori_loop(0, C // L, idx_body, 0)

        cps = [
            pltpu.async_copy(shared.at[i00_v], r00_v, sem),
            pltpu.async_copy(shared.at[i01_v], r01_v, sem),
            pltpu.async_copy(shared.at[i10_v], r10_v, sem),
            pltpu.async_copy(shared.at[i11_v], r11_v, sem),
        ]
        for cp in cps:
            cp.wait()

        def mix_body(i, carry2):
            b = i * L
            iota = lax.iota(jnp.int32, L)
            wx = wx_v[pl.ds(b, L)]
            wy = wy_v[pl.ds(b, L)]
            u00 = plsc.bitcast(r00_v[pl.ds(b, L)], jnp.int32)
            u01 = plsc.bitcast(r01_v[pl.ds(b, L)], jnp.int32)
            u10 = plsc.bitcast(r10_v[pl.ds(b, L)], jnp.int32)
            u11 = plsc.bitcast(r11_v[pl.ds(b, L)], jnp.int32)
            hm = jnp.int32(-65536)
            a00 = plsc.bitcast(lax.shift_left(u00, 16), jnp.float32)
            a01 = plsc.bitcast(lax.shift_left(u01, 16), jnp.float32)
            a10 = plsc.bitcast(lax.shift_left(u10, 16), jnp.float32)
            a11 = plsc.bitcast(lax.shift_left(u11, 16), jnp.float32)
            b00 = plsc.bitcast(u00 & hm, jnp.float32)
            b01 = plsc.bitcast(u01 & hm, jnp.float32)
            b10 = plsc.bitcast(u10 & hm, jnp.float32)
            b11 = plsc.bitcast(u11 & hm, jnp.float32)
            t0 = a00 + wx * (a01 - a00)
            u0 = a10 + wx * (a11 - a10)
            o0 = t0 + wy * (u0 - t0)
            t1 = b00 + wx * (b01 - b00)
            u1 = b10 + wx * (b11 - b10)
            o1 = t1 + wy * (u1 - t1)
            pos = iota * 2 + (2 * b)
            plsc.store_scatter(out_v, [pos], o0)
            plsc.store_scatter(out_v, [pos + 1], o1)
            return carry2

        lax.fori_loop(0, C // L, mix_body, 0)

        pltpu.sync_copy(out_v, out_hbm.at[pl.ds(fbase, 2 * C)])
        return carry

    lax.fori_loop(0, CHUNKS, chunk_body, 0)


_sc_sample = functools.partial(
    pl.kernel,
    out_type=jax.ShapeDtypeStruct((N * FD,), jnp.float32),
    mesh=plsc.VectorSubcoreMesh(
        core_axis_name="c", subcore_axis_name="s", num_cores=NC, num_subcores=NS
    ),
    compiler_params=pltpu.CompilerParams(
        needs_layout_passes=False, use_tc_tiling_on_sc=False),
    scratch_types=[
        pltpu.VMEM_SHARED((PV,), jnp.float32),
        pltpu.VMEM((2 * C,), jnp.float32),
        pltpu.VMEM((C,), jnp.int32),
        pltpu.VMEM((C,), jnp.int32),
        pltpu.VMEM((C,), jnp.int32),
        pltpu.VMEM((C,), jnp.int32),
        pltpu.VMEM((C,), jnp.float32),
        pltpu.VMEM((C,), jnp.float32),
        pltpu.VMEM((C,), jnp.float32),
        pltpu.VMEM((C,), jnp.float32),
        pltpu.VMEM((C,), jnp.float32),
        pltpu.VMEM((C,), jnp.float32),
        pltpu.VMEM((2 * C,), jnp.float32),
        pltpu.SemaphoreType.DMA,
    ],
)(_sc_body)


def kernel(coords, vector_field):
    g16 = lax.bitcast_convert_type(
        vector_field.astype(jnp.bfloat16), jnp.uint16
    ).astype(jnp.uint32)
    packed = g16[..., 0] | (g16[..., 1] << 16)
    packed = jnp.pad(packed, ((0, 1), (0, 1)), mode="edge").reshape(-1)
    packed = jnp.pad(packed, (0, PV - W2 * (H + 1)))
    table = lax.bitcast_convert_type(packed, jnp.float32)
    out = _sc_sample(coords.reshape(-1), table)
    return out.reshape(*coords.shape[:-1], FD)
